# baseline (speedup 1.0000x reference)
import functools

import jax
import jax.numpy as jnp
from jax import lax
from jax.experimental import pallas as pl
from jax.experimental.pallas import tpu as pltpu
from jax.experimental.pallas import tpu_sc as plsc

_C = 256


def _sc_emb(N, H):
    info = plsc.get_sparse_core_info()
    NC, NS = info.num_cores, info.num_subcores
    NW = NC * NS
    b_per_w = N // NW
    n_chunks = b_per_w // _C
    mesh = plsc.VectorSubcoreMesh(core_axis_name="c", subcore_axis_name="s")

    @functools.partial(
        pl.kernel, mesh=mesh,
        out_type=jax.ShapeDtypeStruct((N, H), jnp.float32),
        scratch_types=[
            pltpu.VMEM((_C,), jnp.int32),
            pltpu.VMEM((_C,), jnp.int32),
            pltpu.VMEM((_C, H), jnp.float32),
            pltpu.VMEM((_C, H), jnp.float32),
            pltpu.SemaphoreType.DMA,
        ],
    )
    def k(ymi_hbm, dwi_hbm, ymt_hbm, dwt_hbm, out_hbm,
          ymi_v, dwi_v, ym_rows, dw_rows, sem):
        wid = lax.axis_index("s") * NC + lax.axis_index("c")
        wbase = wid * b_per_w

        def chunk(ci, _):
            base = wbase + ci * _C
            pltpu.sync_copy(ymi_hbm.at[pl.ds(base, _C)], ymi_v)
            pltpu.sync_copy(dwi_hbm.at[pl.ds(base, _C)], dwi_v)
            pltpu.async_copy(ymt_hbm.at[ymi_v], ym_rows, sem).wait()
            pltpu.async_copy(dwt_hbm.at[dwi_v], dw_rows, sem).wait()

            def row(r, _):
                for c in range(H // 16):
                    s = pl.ds(c * 16, 16)
                    ym_rows[r, s] = ym_rows[r, s] + dw_rows[r, s]
                return 0

            lax.fori_loop(0, _C, row, 0)
            pltpu.sync_copy(ym_rows, out_hbm.at[pl.ds(base, _C)])
            return 0

        lax.fori_loop(0, n_chunks, chunk, 0)

    return k


def _tc_body(hol_ref, emb_ref, w_h_ref, b_ref, out_ref, w_ref):
    B = out_ref.shape[1]
    H = out_ref.shape[2]
    i = pl.program_id(0)

    @pl.when(i == 0)
    def _init():
        w_ref[...] = w_h_ref[...].astype(jnp.bfloat16)

    for j in range(out_ref.shape[0]):
        hol = hol_ref[j].astype(jnp.bfloat16)
        lin = jax.lax.dot_general(hol, w_ref[...], (((0,), (1,)), ((), ())),
                                  preferred_element_type=jnp.float32)
        out_ref[j] = emb_ref[j] + lin + b_ref[...]


def kernel(year, month, day, weekday, holidays, year_table, month_table,
           day_table, weekday_table, W_h, b_h):
    B, L = year.shape
    H = year_table.shape[1]
    K = holidays.shape[-1]
    N = B * L

    ymi = (year.T.astype(jnp.int32) * 13 + month.T.astype(jnp.int32)).reshape(N)
    dwi = (day.T.astype(jnp.int32) * 8 + weekday.T.astype(jnp.int32)).reshape(N)
    ymt = (year_table[:, None, :] + month_table[None, :, :]).reshape(-1, H)
    dwt = (day_table[:, None, :] + weekday_table[None, :, :]).reshape(-1, H)

    embsum = _sc_emb(N, H)(ymi, dwi, ymt, dwt).reshape(L, B, H)

    holT = holidays.transpose(1, 2, 0)
    full = lambda shape: pl.BlockSpec(shape, lambda i: tuple(0 for _ in shape))

    outT = pl.pallas_call(
        _tc_body,
        grid=(L // 2,),
        in_specs=[
            pl.BlockSpec((2, K, B), lambda i: (i, 0, 0)),
            pl.BlockSpec((2, B, H), lambda i: (i, 0, 0)),
            full((H, K)),
            full((1, H)),
        ],
        out_specs=pl.BlockSpec((2, B, H), lambda i: (i, 0, 0)),
        out_shape=jax.ShapeDtypeStruct((L, B, H), jnp.float32),
        scratch_shapes=[
            pltpu.VMEM((H, K), jnp.bfloat16),
        ],
        compiler_params=pltpu.CompilerParams(
            dimension_semantics=("arbitrary",),
            fuse_transposed_lhs_in_matmul=True,
        ),
    )(holT, embsum, W_h, b_h.reshape(1, H))
    return outT.transpose(1, 0, 2)


# speedup vs baseline: 5.4448x; 5.4448x over previous
import jax
import jax.numpy as jnp
from jax.experimental import pallas as pl
from jax.experimental.pallas import tpu as pltpu


def _body(p_ref, s_ref, hol_ref, yt_ref, mt_ref, dt_ref, wt_ref, w_h_ref,
          b_ref, out_ref, ct_ref, w_ref):
    B = out_ref.shape[1]
    H = out_ref.shape[2]
    i = pl.program_id(0)

    @pl.when(i == 0)
    def _init():
        ct_ref[0:51] = yt_ref[...].astype(jnp.bfloat16)
        ct_ref[51:64] = mt_ref[...].astype(jnp.bfloat16)
        ct_ref[64:96] = dt_ref[...].astype(jnp.bfloat16)
        ct_ref[96:104] = wt_ref[...].astype(jnp.bfloat16)
        ct_ref[104:128] = jnp.zeros((24, H), jnp.bfloat16)
        w_ref[...] = w_h_ref[...].astype(jnp.bfloat16)

    g = out_ref.shape[0]
    shift = jnp.broadcast_to(s_ref[...], (H, B))
    rows = jax.lax.broadcasted_iota(jnp.int32, (H, B), 0)
    for j in range(g):
        p = p_ref[g * i + j, :].reshape(1, B)
        pb = jnp.broadcast_to(p, (H, B))
        ohT = jnp.where(((pb >> shift) & 127) == rows, 1.0,
                        0.0).astype(jnp.bfloat16)
        emb = jax.lax.dot_general(ohT, ct_ref[...], (((0,), (0,)), ((), ())),
                                  preferred_element_type=jnp.float32)
        hol = hol_ref[j].astype(jnp.bfloat16)
        lin = jax.lax.dot_general(hol, w_ref[...], (((0,), (1,)), ((), ())),
                                  preferred_element_type=jnp.float32)
        out_ref[j] = emb + lin + b_ref[...]


def kernel(year, month, day, weekday, holidays, year_table, month_table,
           day_table, weekday_table, W_h, b_h):
    B, L = year.shape
    H = year_table.shape[1]
    K = holidays.shape[-1]

    packed = (year.T.astype(jnp.int32)
              | ((month.T.astype(jnp.int32) + 51) << 7)
              | ((day.T.astype(jnp.int32) + 64) << 14)
              | ((weekday.T.astype(jnp.int32) + 96) << 21))
    holT = holidays.transpose(1, 2, 0)

    r = jnp.arange(H, dtype=jnp.int32)
    shift = jnp.where(r < 51, 0, jnp.where(r < 64, 7,
                      jnp.where(r < 96, 14, 21))).reshape(H, 1)

    full = lambda shape: pl.BlockSpec(shape, lambda i: tuple(0 for _ in shape))

    outT = pl.pallas_call(
        _body,
        grid=(L // 5,),
        in_specs=[
            full((L, B)),
            full((H, 1)),
            pl.BlockSpec((5, K, B), lambda i: (i, 0, 0)),
            full((51, H)),
            full((13, H)),
            full((32, H)),
            full((8, H)),
            full((H, K)),
            full((1, H)),
        ],
        out_specs=pl.BlockSpec((5, B, H), lambda i: (i, 0, 0)),
        out_shape=jax.ShapeDtypeStruct((L, B, H), jnp.float32),
        scratch_shapes=[
            pltpu.VMEM((128, H), jnp.bfloat16),
            pltpu.VMEM((H, K), jnp.bfloat16),
        ],
        compiler_params=pltpu.CompilerParams(
            dimension_semantics=("arbitrary",),
            fuse_transposed_lhs_in_matmul=True,
        ),
    )(packed, shift, holT, year_table, month_table, day_table, weekday_table,
      W_h, b_h.reshape(1, H))
    return outT.transpose(1, 0, 2)
